# baseline (device time: 40034 ns/iter reference)
import jax
import jax.numpy as jnp
from jax import lax
from jax.experimental import pallas as pl
from jax.experimental.pallas import tpu as pltpu

N_DEV = 4
N_TOK = 1024
D_MODEL = 256
D_HID = 512
N_EXP = 16
E_LOCAL = N_EXP // N_DEV
M_PER = N_TOK // N_DEV


def kernel(x, router_W, route_idx, expert_W):
    def body(x_ref, rw_ref, idx_ref, ew_ref, out_ref,
             acc_ref, recv_ref, send_sems, recv_sems):
        my_pos = lax.axis_index("i")
        left = lax.rem(my_pos + N_DEV - 1, N_DEV)
        right = lax.rem(my_pos + 1, N_DEV)

        barrier_sem = pltpu.get_barrier_semaphore()
        for nbr in (left, right):
            pl.semaphore_signal(
                barrier_sem, inc=1,
                device_id=(nbr,), device_id_type=pl.DeviceIdType.MESH,
            )
        pl.semaphore_wait(barrier_sem, 2)

        xv = x_ref[:, :]
        scores = jnp.dot(xv, rw_ref[:, :],
                         preferred_element_type=jnp.float32,
                         precision=lax.Precision.HIGHEST)
        s_max = jnp.max(scores, axis=-1, keepdims=True)
        e = jnp.exp(scores - s_max)
        probs = e / jnp.sum(e, axis=-1, keepdims=True)

        idx = idx_ref[:, :]
        e_iota = lax.broadcasted_iota(jnp.int32, (N_TOK, N_EXP), 1)
        hit0 = idx[:, 0:1] == e_iota
        hit1 = idx[:, 1:2] == e_iota
        g0 = jnp.sum(jnp.where(hit0, probs, 0.0), axis=-1, keepdims=True)
        g1 = jnp.sum(jnp.where(hit1, probs, 0.0), axis=-1, keepdims=True)
        denom = g0 + g1
        w = jnp.where(hit0 | hit1, probs, 0.0) / denom

        acc = jnp.zeros((N_TOK, D_HID), jnp.float32)
        for j in range(E_LOCAL):
            ge = my_pos * E_LOCAL + j
            coeff = jnp.sum(jnp.where(e_iota == ge, w, 0.0),
                            axis=-1, keepdims=True)
            acc = acc + jnp.dot(xv * coeff, ew_ref[j],
                                preferred_element_type=jnp.float32,
                                precision=lax.Precision.HIGHEST)
        acc_ref[:, :] = acc

        for s in range(N_DEV - 1):
            send_blk = lax.rem(my_pos + 2 * N_DEV - s - 1, N_DEV)
            recv_blk = lax.rem(my_pos + 2 * N_DEV - s - 2, N_DEV)
            rdma = pltpu.make_async_remote_copy(
                src_ref=acc_ref.at[pl.ds(send_blk * M_PER, M_PER), :],
                dst_ref=recv_ref.at[s],
                send_sem=send_sems.at[s],
                recv_sem=recv_sems.at[s],
                device_id=(right,),
                device_id_type=pl.DeviceIdType.MESH,
            )
            rdma.start()
            rdma.wait()
            rs = recv_blk * M_PER
            acc_ref[pl.ds(rs, M_PER), :] = (
                acc_ref[pl.ds(rs, M_PER), :] + recv_ref[s]
            )

        out_ref[:, :] = acc_ref[pl.ds(my_pos * M_PER, M_PER), :]

    return pl.pallas_call(
        body,
        out_shape=jax.ShapeDtypeStruct((M_PER, D_HID), jnp.float32),
        in_specs=[
            pl.BlockSpec(memory_space=pltpu.VMEM),
            pl.BlockSpec(memory_space=pltpu.VMEM),
            pl.BlockSpec(memory_space=pltpu.VMEM),
            pl.BlockSpec(memory_space=pltpu.VMEM),
        ],
        out_specs=pl.BlockSpec(memory_space=pltpu.VMEM),
        scratch_shapes=[
            pltpu.VMEM((N_TOK, D_HID), jnp.float32),
            pltpu.VMEM((N_DEV - 1, M_PER, D_HID), jnp.float32),
            pltpu.SemaphoreType.DMA((N_DEV - 1,)),
            pltpu.SemaphoreType.DMA((N_DEV - 1,)),
        ],
        compiler_params=pltpu.CompilerParams(collective_id=0),
    )(x, router_W, route_idx, expert_W)


# device time: 30723 ns/iter; 1.3031x vs baseline; 1.3031x over previous
import jax
import jax.numpy as jnp
from jax import lax
from jax.experimental import pallas as pl
from jax.experimental.pallas import tpu as pltpu

N_DEV = 4
N_TOK = 1024
D_MODEL = 256
D_HID = 512
N_EXP = 16
E_LOCAL = N_EXP // N_DEV
M_PER = N_TOK // N_DEV


def kernel(x, router_W, route_idx, expert_W):
    def body(x_ref, rw_ref, idx_ref, ew_ref, out_ref,
             w_ref, send_ref, recv_ref, send_sems, recv_sems):
        my_pos = lax.axis_index("i")

        barrier_sem = pltpu.get_barrier_semaphore()
        for r in range(1, N_DEV):
            pl.semaphore_signal(
                barrier_sem, inc=1,
                device_id=(lax.rem(my_pos + r, N_DEV),),
                device_id_type=pl.DeviceIdType.MESH,
            )
        pl.semaphore_wait(barrier_sem, N_DEV - 1)

        scores = jnp.dot(x_ref[:, :], rw_ref[:, :],
                         preferred_element_type=jnp.float32,
                         precision=lax.Precision.HIGHEST)
        s_max = jnp.max(scores, axis=-1, keepdims=True)
        e = jnp.exp(scores - s_max)
        probs = e / jnp.sum(e, axis=-1, keepdims=True)

        idx = idx_ref[:, :]
        e_iota = lax.broadcasted_iota(jnp.int32, (N_TOK, N_EXP), 1)
        hit0 = idx[:, 0:1] == e_iota
        hit1 = idx[:, 1:2] == e_iota
        g0 = jnp.sum(jnp.where(hit0, probs, 0.0), axis=-1, keepdims=True)
        g1 = jnp.sum(jnp.where(hit1, probs, 0.0), axis=-1, keepdims=True)
        w_ref[:, :] = jnp.where(hit0 | hit1, probs, 0.0) / (g0 + g1)

        blk_iota = lax.broadcasted_iota(jnp.int32, (M_PER, N_EXP), 1)

        def block_partial(b):
            rs = b * M_PER
            xb = x_ref[pl.ds(rs, M_PER), :]
            wb = w_ref[pl.ds(rs, M_PER), :]
            blk = jnp.zeros((M_PER, D_HID), jnp.float32)
            for j in range(E_LOCAL):
                ge = my_pos * E_LOCAL + j
                coeff = jnp.sum(jnp.where(blk_iota == ge, wb, 0.0),
                                axis=-1, keepdims=True)
                blk = blk + jnp.dot(xb * coeff, ew_ref[j],
                                    preferred_element_type=jnp.float32,
                                    precision=lax.Precision.HIGHEST)
            return blk

        rdmas = []
        for r in range(1, N_DEV):
            dst = lax.rem(my_pos + r, N_DEV)
            send_ref[r - 1, :, :] = block_partial(dst)
            rdma = pltpu.make_async_remote_copy(
                src_ref=send_ref.at[r - 1],
                dst_ref=recv_ref.at[r - 1],
                send_sem=send_sems.at[r - 1],
                recv_sem=recv_sems.at[r - 1],
                device_id=(dst,),
                device_id_type=pl.DeviceIdType.MESH,
            )
            rdma.start()
            rdmas.append(rdma)

        total = block_partial(my_pos)
        for rdma in rdmas:
            rdma.wait_recv()
        for r in range(1, N_DEV):
            total = total + recv_ref[r - 1, :, :]
        out_ref[:, :] = total

        for rdma in rdmas:
            rdma.wait_send()

    return pl.pallas_call(
        body,
        out_shape=jax.ShapeDtypeStruct((M_PER, D_HID), jnp.float32),
        in_specs=[
            pl.BlockSpec(memory_space=pltpu.VMEM),
            pl.BlockSpec(memory_space=pltpu.VMEM),
            pl.BlockSpec(memory_space=pltpu.VMEM),
            pl.BlockSpec(memory_space=pltpu.VMEM),
        ],
        out_specs=pl.BlockSpec(memory_space=pltpu.VMEM),
        scratch_shapes=[
            pltpu.VMEM((N_TOK, N_EXP), jnp.float32),
            pltpu.VMEM((N_DEV - 1, M_PER, D_HID), jnp.float32),
            pltpu.VMEM((N_DEV - 1, M_PER, D_HID), jnp.float32),
            pltpu.SemaphoreType.DMA((N_DEV - 1,)),
            pltpu.SemaphoreType.DMA((N_DEV - 1,)),
        ],
        compiler_params=pltpu.CompilerParams(collective_id=0),
    )(x, router_W, route_idx, expert_W)


# device time: 19911 ns/iter; 2.0106x vs baseline; 1.5430x over previous
import jax
import jax.numpy as jnp
from jax import lax
from jax.experimental import pallas as pl
from jax.experimental.pallas import tpu as pltpu

N_DEV = 4
N_TOK = 1024
D_MODEL = 256
D_HID = 512
N_EXP = 16
E_LOCAL = N_EXP // N_DEV
M_PER = N_TOK // N_DEV


def kernel(x, router_W, route_idx, expert_W):
    def body(x_ref, rw_ref, idx_ref, ew_ref, out_ref,
             w_ref, send_ref, recv_ref, send_sems, recv_sems):
        my_pos = lax.axis_index("i")

        barrier_sem = pltpu.get_barrier_semaphore()
        for r in range(1, N_DEV):
            pl.semaphore_signal(
                barrier_sem, inc=1,
                device_id=(lax.rem(my_pos + r, N_DEV),),
                device_id_type=pl.DeviceIdType.MESH,
            )
        pl.semaphore_wait(barrier_sem, N_DEV - 1)

        scores = jnp.dot(x_ref[:, :], rw_ref[:, :],
                         preferred_element_type=jnp.float32,
                         precision=lax.Precision.HIGHEST)
        s_max = jnp.max(scores, axis=-1, keepdims=True)
        e = jnp.exp(scores - s_max)
        probs = e / jnp.sum(e, axis=-1, keepdims=True)

        idx = idx_ref[:, :]
        e_iota = lax.broadcasted_iota(jnp.int32, (N_TOK, N_EXP), 1)
        hit0 = idx[:, 0:1] == e_iota
        hit1 = idx[:, 1:2] == e_iota
        g0 = jnp.sum(jnp.where(hit0, probs, 0.0), axis=-1, keepdims=True)
        g1 = jnp.sum(jnp.where(hit1, probs, 0.0), axis=-1, keepdims=True)
        w_ref[:, :] = jnp.where(hit0 | hit1, probs, 0.0) / (g0 + g1)

        blk_iota = lax.broadcasted_iota(jnp.int32, (M_PER, N_EXP), 1)

        def block_partial(b):
            rs = b * M_PER
            xb = x_ref[pl.ds(rs, M_PER), :]
            wb = w_ref[pl.ds(rs, M_PER), :]
            blk = jnp.zeros((M_PER, D_HID), jnp.float32)
            for j in range(E_LOCAL):
                ge = my_pos * E_LOCAL + j
                coeff = jnp.sum(jnp.where(blk_iota == ge, wb, 0.0),
                                axis=-1, keepdims=True)
                blk = blk + jnp.dot((xb * coeff).astype(jnp.bfloat16),
                                    ew_ref[j].astype(jnp.bfloat16),
                                    preferred_element_type=jnp.float32)
            return blk

        rdmas = []
        for r in range(1, N_DEV):
            dst = lax.rem(my_pos + r, N_DEV)
            send_ref[r - 1, :, :] = block_partial(dst).astype(jnp.bfloat16)
            rdma = pltpu.make_async_remote_copy(
                src_ref=send_ref.at[r - 1],
                dst_ref=recv_ref.at[r - 1],
                send_sem=send_sems.at[r - 1],
                recv_sem=recv_sems.at[r - 1],
                device_id=(dst,),
                device_id_type=pl.DeviceIdType.MESH,
            )
            rdma.start()
            rdmas.append(rdma)

        total = block_partial(my_pos)
        for rdma in rdmas:
            rdma.wait_recv()
        for r in range(1, N_DEV):
            total = total + recv_ref[r - 1, :, :].astype(jnp.float32)
        out_ref[:, :] = total

        for rdma in rdmas:
            rdma.wait_send()

    return pl.pallas_call(
        body,
        out_shape=jax.ShapeDtypeStruct((M_PER, D_HID), jnp.float32),
        in_specs=[
            pl.BlockSpec(memory_space=pltpu.VMEM),
            pl.BlockSpec(memory_space=pltpu.VMEM),
            pl.BlockSpec(memory_space=pltpu.VMEM),
            pl.BlockSpec(memory_space=pltpu.VMEM),
        ],
        out_specs=pl.BlockSpec(memory_space=pltpu.VMEM),
        scratch_shapes=[
            pltpu.VMEM((N_TOK, N_EXP), jnp.float32),
            pltpu.VMEM((N_DEV - 1, M_PER, D_HID), jnp.bfloat16),
            pltpu.VMEM((N_DEV - 1, M_PER, D_HID), jnp.bfloat16),
            pltpu.SemaphoreType.DMA((N_DEV - 1,)),
            pltpu.SemaphoreType.DMA((N_DEV - 1,)),
        ],
        compiler_params=pltpu.CompilerParams(collective_id=0),
    )(x, router_W, route_idx, expert_W)


# device time: 10050 ns/iter; 3.9835x vs baseline; 1.9812x over previous
import jax
import jax.numpy as jnp
from jax import lax
from jax.experimental import pallas as pl
from jax.experimental.pallas import tpu as pltpu

N_DEV = 4
N_TOK = 1024
D_MODEL = 256
D_HID = 512
N_EXP = 16
E_LOCAL = N_EXP // N_DEV
M_PER = N_TOK // N_DEV


def kernel(x, router_W, route_idx, expert_W):
    def body(x_ref, rw_ref, idx_ref, ew_ref, out_ref,
             w_ref, send_ref, recv_ref, send_sems, recv_sems):
        my_pos = lax.axis_index("i")

        barrier_sem = pltpu.get_barrier_semaphore()
        for r in range(1, N_DEV):
            pl.semaphore_signal(
                barrier_sem, inc=1,
                device_id=(lax.rem(my_pos + r, N_DEV),),
                device_id_type=pl.DeviceIdType.MESH,
            )
        pl.semaphore_wait(barrier_sem, N_DEV - 1)

        scores = jnp.dot(x_ref[:, :], rw_ref[:, :],
                         preferred_element_type=jnp.float32,
                         precision=lax.Precision.HIGHEST)
        s_max = jnp.max(scores, axis=-1, keepdims=True)
        e = jnp.exp(scores - s_max)
        probs = e / jnp.sum(e, axis=-1, keepdims=True)

        idx = idx_ref[:, :]
        e_iota = lax.broadcasted_iota(jnp.int32, (N_TOK, N_EXP), 1)
        hit0 = idx[:, 0:1] == e_iota
        hit1 = idx[:, 1:2] == e_iota
        g0 = jnp.sum(jnp.where(hit0, probs, 0.0), axis=-1, keepdims=True)
        g1 = jnp.sum(jnp.where(hit1, probs, 0.0), axis=-1, keepdims=True)
        w_ref[:, :] = jnp.where(hit0 | hit1, probs, 0.0) / (g0 + g1)

        blk_iota = lax.broadcasted_iota(jnp.int32, (M_PER, N_EXP), 1)
        eww = ew_ref[:, :, :].astype(jnp.bfloat16).reshape(
            E_LOCAL * D_MODEL, D_HID)

        def block_partial(b):
            rs = b * M_PER
            xb = x_ref[pl.ds(rs, M_PER), :]
            wb = w_ref[pl.ds(rs, M_PER), :]
            parts = []
            for j in range(E_LOCAL):
                ge = my_pos * E_LOCAL + j
                coeff = jnp.sum(jnp.where(blk_iota == ge, wb, 0.0),
                                axis=-1, keepdims=True)
                parts.append((xb * coeff).astype(jnp.bfloat16))
            return jnp.dot(jnp.concatenate(parts, axis=1), eww,
                           preferred_element_type=jnp.float32)

        rdmas = []
        for r in range(1, N_DEV):
            dst = lax.rem(my_pos + r, N_DEV)
            send_ref[r - 1, :, :] = block_partial(dst).astype(jnp.bfloat16)
            rdma = pltpu.make_async_remote_copy(
                src_ref=send_ref.at[r - 1],
                dst_ref=recv_ref.at[r - 1],
                send_sem=send_sems.at[r - 1],
                recv_sem=recv_sems.at[r - 1],
                device_id=(dst,),
                device_id_type=pl.DeviceIdType.MESH,
            )
            rdma.start()
            rdmas.append(rdma)

        total = block_partial(my_pos)
        for rdma in rdmas:
            rdma.wait_recv()
        for r in range(1, N_DEV):
            total = total + recv_ref[r - 1, :, :].astype(jnp.float32)
        out_ref[:, :] = total

        for rdma in rdmas:
            rdma.wait_send()

    return pl.pallas_call(
        body,
        out_shape=jax.ShapeDtypeStruct((M_PER, D_HID), jnp.float32),
        in_specs=[
            pl.BlockSpec(memory_space=pltpu.VMEM),
            pl.BlockSpec(memory_space=pltpu.VMEM),
            pl.BlockSpec(memory_space=pltpu.VMEM),
            pl.BlockSpec(memory_space=pltpu.VMEM),
        ],
        out_specs=pl.BlockSpec(memory_space=pltpu.VMEM),
        scratch_shapes=[
            pltpu.VMEM((N_TOK, N_EXP), jnp.float32),
            pltpu.VMEM((N_DEV - 1, M_PER, D_HID), jnp.bfloat16),
            pltpu.VMEM((N_DEV - 1, M_PER, D_HID), jnp.bfloat16),
            pltpu.SemaphoreType.DMA((N_DEV - 1,)),
            pltpu.SemaphoreType.DMA((N_DEV - 1,)),
        ],
        compiler_params=pltpu.CompilerParams(collective_id=0),
    )(x, router_W, route_idx, expert_W)
